# Initial kernel scaffold; baseline (speedup 1.0000x reference)
import jax
import jax.numpy as jnp
from jax.experimental import pallas as pl


def kernel(x):
    raise NotImplementedError("write your pallas kernel here")


# speedup vs baseline: 2.1152x; 2.1152x over previous
import jax
import jax.numpy as jnp
from jax import lax
from jax.experimental import pallas as pl

NUM_CLS = 1000
ROWS_PER_BLOCK = 1024


def _body(x_ref, o_ref):
    xb = x_ref[0, 0, :]
    cols = lax.broadcasted_iota(jnp.int32, (ROWS_PER_BLOCK, NUM_CLS), 1)
    o_ref[...] = (xb[:, None] == cols).astype(jnp.float32)


def kernel(x):
    B = x.shape[0]
    nb = B // ROWS_PER_BLOCK
    x3 = x.reshape(nb, 1, ROWS_PER_BLOCK)
    return pl.pallas_call(
        _body,
        grid=(nb,),
        in_specs=[pl.BlockSpec((1, 1, ROWS_PER_BLOCK), lambda i: (i, 0, 0))],
        out_specs=pl.BlockSpec((ROWS_PER_BLOCK, NUM_CLS), lambda i: (i, 0)),
        out_shape=jax.ShapeDtypeStruct((B, NUM_CLS), jnp.float32),
    )(x3)
